# baseline (speedup 1.0000x reference)
import functools

import jax
import jax.numpy as jnp
from jax import lax
from jax.experimental import pallas as pl
from jax.experimental.pallas import tpu as pltpu
from jax.experimental.pallas import tpu_sc as plsc

NN = 4096
DD = 256
HH = 256
OUT = 64
BLK = 512
NBK = NN // BLK
NT = NBK * (NBK + 1) // 2
UROWS = NT * BLK
TOT = NN * NN
K_RANK = int(round(0.99 * (TOT - 1)))


def _tri_base(a):
    return a * (2 * NBK + 1 - a) // 2


def _tri_a(t):
    a = jnp.int32(0)
    for k in range(1, NBK):
        a = a + (t >= _tri_base(k)).astype(jnp.int32)
    return a


NC, NS, L = 2, 16, 16
NW = NC * NS
ROWS_PW = UROWS // NW
CR = 32
VPR = BLK // L
NB1 = 1 << 15
NB2 = 1 << 16


@functools.cache
def _sc_mesh():
    return plsc.VectorSubcoreMesh(
        core_axis_name="c", subcore_axis_name="s",
        num_cores=NC, num_subcores=NS)


def _zero_hist(hist, nb):
    zeros = jnp.zeros((L,), jnp.int32)

    def body(t, carry):
        hist[pl.ds(t * L, L)] = zeros
        return carry

    lax.fori_loop(0, nb // L, body, 0)


_DIAG_TS = tuple(_tri_base(a) for a in range(NBK))


def _chunk_weight(row):
    tt = jax.lax.shift_right_logical(row, 9)
    isd = jnp.int32(0)
    for t in _DIAG_TS:
        isd = isd | (tt == t).astype(jnp.int32)
    return jnp.broadcast_to(2 - isd, (L,))


def _scan_rows(u_hbm, row0, buf0, buf1, sem0, sem1, process):
    nch = ROWS_PW // CR

    def start(c, buf, sem):
        pltpu.make_async_copy(
            u_hbm.at[pl.ds(row0 + c * CR, CR)], buf, sem).start()

    def wait(buf, sem):
        pltpu.make_async_copy(u_hbm.at[pl.ds(row0, CR)], buf, sem).wait()

    start(0, buf0, sem0)

    def outer(g, carry):
        c0 = 2 * g
        start(c0 + 1, buf1, sem1)
        wait(buf0, sem0)
        process(buf0, row0 + c0 * CR)

        @pl.when(c0 + 2 < nch)
        def _():
            start(c0 + 2, buf0, sem0)

        wait(buf1, sem1)
        process(buf1, row0 + (c0 + 1) * CR)
        return carry

    lax.fori_loop(0, nch // 2, outer, 0)


def _hist1_body(u_hbm, out_hbm, buf0, buf1, hist, sem0, sem1):
    wid = lax.axis_index("s") * NC + lax.axis_index("c")
    _zero_hist(hist, NB1)

    def process(buf, row):
        w = _chunk_weight(row)
        for r in range(CR):
            @plsc.parallel_loop(0, VPR, unroll=8)
            def _(t):
                v = buf[r, pl.ds(t * L, L)]
                bits = lax.bitcast_convert_type(v, jnp.int32)
                idx = jax.lax.shift_right_logical(bits, 16)
                plsc.addupdate_scatter(hist, [idx], w)

    _scan_rows(u_hbm, wid * ROWS_PW, buf0, buf1, sem0, sem1, process)
    pltpu.sync_copy(hist, out_hbm.at[wid])


def _hist2_body(u_hbm, b1_hbm, out_hbm, buf0, buf1, hist, b1v, sem0, sem1):
    wid = lax.axis_index("s") * NC + lax.axis_index("c")
    _zero_hist(hist, NB2)
    pltpu.sync_copy(b1_hbm, b1v)
    b1 = b1v[...]

    def process(buf, row):
        w = _chunk_weight(row)
        for r in range(CR):
            @plsc.parallel_loop(0, VPR, unroll=8)
            def _(t):
                v = buf[r, pl.ds(t * L, L)]
                bits = lax.bitcast_convert_type(v, jnp.int32)
                hi = jax.lax.shift_right_logical(bits, 16)
                lo = jnp.bitwise_and(bits, 0xFFFF)
                plsc.addupdate_scatter(hist, [lo], w, mask=hi == b1)

    _scan_rows(u_hbm, wid * ROWS_PW, buf0, buf1, sem0, sem1, process)
    pltpu.sync_copy(hist, out_hbm.at[wid])


@functools.cache
def _hist1():
    return pl.kernel(
        _hist1_body,
        out_type=jax.ShapeDtypeStruct((NW, NB1), jnp.int32),
        mesh=_sc_mesh(),
        scratch_types=[
            pltpu.VMEM((CR, BLK), jnp.float32),
            pltpu.VMEM((CR, BLK), jnp.float32),
            pltpu.VMEM((NB1,), jnp.int32),
            pltpu.SemaphoreType.DMA,
            pltpu.SemaphoreType.DMA,
        ],
        compiler_params=pltpu.CompilerParams(needs_layout_passes=False),
    )


@functools.cache
def _hist2():
    return pl.kernel(
        _hist2_body,
        out_type=jax.ShapeDtypeStruct((NW, NB2), jnp.int32),
        mesh=_sc_mesh(),
        scratch_types=[
            pltpu.VMEM((CR, BLK), jnp.float32),
            pltpu.VMEM((CR, BLK), jnp.float32),
            pltpu.VMEM((NB2,), jnp.int32),
            pltpu.VMEM((L,), jnp.int32),
            pltpu.SemaphoreType.DMA,
            pltpu.SemaphoreType.DMA,
        ],
        compiler_params=pltpu.CompilerParams(needs_layout_passes=False),
    )


def _prep_body(x_ref, win_ref, xn_ref, h_ref, hb_ref):
    x = x_ref[...]
    nrm = jnp.sqrt(jnp.sum(x * x, axis=1, keepdims=True))
    xn_ref[...] = x / jnp.maximum(nrm, 1e-8)
    h = lax.dot_general(x, win_ref[...], (((1,), (1,)), ((), ())),
                        preferred_element_type=jnp.float32)
    h = jnp.maximum(h, 0.0)
    h_ref[...] = h
    hb_ref[...] = h.astype(jnp.bfloat16)


def _sim_body(xi_ref, xj_ref, sim_ref):
    sim_ref[...] = jnp.abs(
        lax.dot_general(xi_ref[...], xj_ref[...], (((1,), (1,)), ((), ())),
                        preferred_element_type=jnp.float32))


def _sage_body(eps_ref, u_ref, hba_ref, hbb_ref, hfull_ref, wl_ref, bl_ref,
               wr_ref, wo_ref, bo_ref, out_ref, acc_ref, cnt_ref):
    t = pl.program_id(0)
    a = _tri_a(t)
    b = t - a * (2 * NBK + 1 - a) // 2 + a

    @pl.when(t == 0)
    def _():
        acc_ref[...] = jnp.zeros_like(acc_ref)
        cnt_ref[...] = jnp.zeros_like(cnt_ref)

    eps = eps_ref[0, 0]
    mask = u_ref[...] >= eps
    a16 = mask.astype(jnp.bfloat16)

    acc_ref[pl.ds(a * BLK, BLK), :] += lax.dot_general(
        a16, hbb_ref[...], (((1,), (0,)), ((), ())),
        preferred_element_type=jnp.float32)
    csum = jnp.sum(mask.astype(jnp.float32), axis=1, keepdims=True)
    cnt_ref[pl.ds(a * BLK, BLK), :] += jnp.broadcast_to(
        csum, (BLK, cnt_ref.shape[1]))

    @pl.when(b != a)
    def _():
        acc_ref[pl.ds(b * BLK, BLK), :] += lax.dot_general(
            a16, hba_ref[...], (((0,), (0,)), ((), ())),
            preferred_element_type=jnp.float32)
        ones = jnp.ones((BLK, cnt_ref.shape[1]), jnp.bfloat16)
        cnt_ref[pl.ds(b * BLK, BLK), :] += lax.dot_general(
            a16, ones, (((0,), (0,)), ((), ())),
            preferred_element_type=jnp.float32)

    @pl.when(t == pl.num_programs(0) - 1)
    def _():
        acc = acc_ref[...]
        cnt = cnt_ref[...][:, :1]
        mean = acc / jnp.maximum(cnt, 1.0)
        h = hfull_ref[...]
        z = (lax.dot_general(mean, wl_ref[...], (((1,), (1,)), ((), ())),
                             preferred_element_type=jnp.float32)
             + bl_ref[...]
             + lax.dot_general(h, wr_ref[...], (((1,), (1,)), ((), ())),
                               preferred_element_type=jnp.float32))
        z = jnp.maximum(z, 0.0)
        o = lax.dot_general(z, wo_ref[...], (((1,), (1,)), ((), ())),
                            preferred_element_type=jnp.float32) + bo_ref[...]
        m = jnp.max(o, axis=1, keepdims=True)
        e = jnp.exp(o - m)
        out_ref[...] = e / jnp.sum(e, axis=1, keepdims=True)


def _prep_call(x, W_in):
    return pl.pallas_call(
        _prep_body,
        grid=(NBK,),
        in_specs=[
            pl.BlockSpec((BLK, DD), lambda i: (i, 0)),
            pl.BlockSpec((HH, DD), lambda i: (0, 0)),
        ],
        out_specs=[
            pl.BlockSpec((BLK, DD), lambda i: (i, 0)),
            pl.BlockSpec((BLK, HH), lambda i: (i, 0)),
            pl.BlockSpec((BLK, HH), lambda i: (i, 0)),
        ],
        out_shape=[
            jax.ShapeDtypeStruct((NN, DD), jnp.float32),
            jax.ShapeDtypeStruct((NN, HH), jnp.float32),
            jax.ShapeDtypeStruct((NN, HH), jnp.bfloat16),
        ],
    )(x, W_in)


def _sim_call(xn):
    def imap_i(t):
        return (_tri_a(t), 0)

    def imap_j(t):
        a = _tri_a(t)
        return (t - a * (2 * NBK + 1 - a) // 2 + a, 0)

    return pl.pallas_call(
        _sim_body,
        grid=(NT,),
        in_specs=[
            pl.BlockSpec((BLK, DD), imap_i),
            pl.BlockSpec((BLK, DD), imap_j),
        ],
        out_specs=pl.BlockSpec((BLK, BLK), lambda t: (t, 0)),
        out_shape=jax.ShapeDtypeStruct((UROWS, BLK), jnp.float32),
    )(xn, xn)


def _select_eps(h1, h2_fn):
    c1 = jnp.sum(h1, axis=0)
    cum1 = jnp.cumsum(c1)
    b1 = jnp.argmax(cum1 >= (K_RANK + 1)).astype(jnp.int32)
    rank2 = K_RANK - (cum1[b1] - c1[b1])
    h2 = h2_fn(jnp.full((L,), b1, jnp.int32))
    c2 = jnp.sum(h2, axis=0)
    cum2 = jnp.cumsum(c2)
    b2 = jnp.argmax(cum2 >= (rank2 + 1)).astype(jnp.int32)
    eps_bits = jnp.bitwise_or(jnp.left_shift(b1, 16), b2)
    return lax.bitcast_convert_type(eps_bits, jnp.float32)


def _sage_call(eps, u, h, hb, Wl, bl, Wr, Wo, bo):
    def imap_a(t):
        return (_tri_a(t), 0)

    def imap_b(t):
        a = _tri_a(t)
        return (t - a * (2 * NBK + 1 - a) // 2 + a, 0)

    return pl.pallas_call(
        _sage_body,
        grid=(NT,),
        in_specs=[
            pl.BlockSpec(memory_space=pltpu.SMEM),
            pl.BlockSpec((BLK, BLK), lambda t: (t, 0)),
            pl.BlockSpec((BLK, HH), imap_a),
            pl.BlockSpec((BLK, HH), imap_b),
            pl.BlockSpec((NN, HH), lambda t: (0, 0)),
            pl.BlockSpec((HH, HH), lambda t: (0, 0)),
            pl.BlockSpec((1, HH), lambda t: (0, 0)),
            pl.BlockSpec((HH, HH), lambda t: (0, 0)),
            pl.BlockSpec((OUT, HH), lambda t: (0, 0)),
            pl.BlockSpec((1, OUT), lambda t: (0, 0)),
        ],
        out_specs=pl.BlockSpec((NN, OUT), lambda t: (0, 0)),
        out_shape=jax.ShapeDtypeStruct((NN, OUT), jnp.float32),
        scratch_shapes=[
            pltpu.VMEM((NN, HH), jnp.float32),
            pltpu.VMEM((NN, 128), jnp.float32),
        ],
    )(eps, u, hb, hb, h, Wl, bl.reshape(1, HH), Wr, Wo, bo.reshape(1, OUT))


def kernel(x, W_in, Wl, bl, Wr, Wo, bo):
    xn, h, hb = _prep_call(x, W_in)
    u = _sim_call(xn)
    h1 = _hist1()(u)
    eps = _select_eps(h1, lambda b1v: _hist2()(u, b1v))
    return _sage_call(eps.reshape(1, 1), u, h, hb, Wl, bl, Wr, Wo, bo)


# speedup vs baseline: 1.1446x; 1.1446x over previous
import functools

import jax
import jax.numpy as jnp
from jax import lax
from jax.experimental import pallas as pl
from jax.experimental.pallas import tpu as pltpu
from jax.experimental.pallas import tpu_sc as plsc

NN = 4096
DD = 256
HH = 256
OUT = 64
BLK = 512
NBK = NN // BLK
NT = NBK * (NBK + 1) // 2
UROWS = NT * BLK
TOT = NN * NN
K_RANK = int(round(0.99 * (TOT - 1)))


def _tri_base(a):
    return a * (2 * NBK + 1 - a) // 2


def _tri_a(t):
    a = jnp.int32(0)
    for k in range(1, NBK):
        a = a + (t >= _tri_base(k)).astype(jnp.int32)
    return a


NC, NS, L = 2, 16, 16
NW = NC * NS
ROWS_PW = UROWS // NW
CR = 32
VPR = BLK // L
NB1 = 1 << 15
NB2 = 1 << 16


@functools.cache
def _sc_mesh():
    return plsc.VectorSubcoreMesh(
        core_axis_name="c", subcore_axis_name="s",
        num_cores=NC, num_subcores=NS)


def _zero_hist(hist, nb):
    zeros = jnp.zeros((L,), jnp.int32)

    def body(t, carry):
        hist[pl.ds(t * L, L)] = zeros
        return carry

    lax.fori_loop(0, nb // L, body, 0)


_DIAG_TS = tuple(_tri_base(a) for a in range(NBK))


def _chunk_weight(row):
    tt = jax.lax.shift_right_logical(row, 9)
    isd = jnp.int32(0)
    for t in _DIAG_TS:
        isd = isd | (tt == t).astype(jnp.int32)
    return jnp.broadcast_to(2 - isd, (L,))


def _scan_rows(u_hbm, row0, buf0, buf1, sem0, sem1, process):
    nch = ROWS_PW // CR

    def start(c, buf, sem):
        pltpu.make_async_copy(
            u_hbm.at[pl.ds(row0 + c * CR, CR)], buf, sem).start()

    def wait(buf, sem):
        pltpu.make_async_copy(u_hbm.at[pl.ds(row0, CR)], buf, sem).wait()

    start(0, buf0, sem0)

    def outer(g, carry):
        c0 = 2 * g
        start(c0 + 1, buf1, sem1)
        wait(buf0, sem0)
        process(buf0, row0 + c0 * CR)

        @pl.when(c0 + 2 < nch)
        def _():
            start(c0 + 2, buf0, sem0)

        wait(buf1, sem1)
        process(buf1, row0 + (c0 + 1) * CR)
        return carry

    lax.fori_loop(0, nch // 2, outer, 0)


def _hist1_body(u_hbm, out_hbm, buf0, buf1, hist, sem0, sem1):
    wid = lax.axis_index("s") * NC + lax.axis_index("c")
    _zero_hist(hist, NB1)

    def process(buf, row):
        w = _chunk_weight(row)

        @plsc.parallel_loop(0, CR * VPR, unroll=8)
        def _(t):
            rr = jax.lax.shift_right_logical(t, 5)
            cc = jnp.bitwise_and(t, VPR - 1)
            v = buf[rr, pl.ds(cc * L, L)]
            bits = lax.bitcast_convert_type(v, jnp.int32)
            idx = jax.lax.shift_right_logical(bits, 16)
            plsc.addupdate_scatter(hist, [idx], w)

    _scan_rows(u_hbm, wid * ROWS_PW, buf0, buf1, sem0, sem1, process)
    pltpu.sync_copy(hist, out_hbm.at[wid])


def _hist2_body(u_hbm, b1_hbm, out_hbm, buf0, buf1, hist, b1v, sem0, sem1):
    wid = lax.axis_index("s") * NC + lax.axis_index("c")
    _zero_hist(hist, NB2)
    pltpu.sync_copy(b1_hbm, b1v)
    b1 = b1v[...]

    def process(buf, row):
        w = _chunk_weight(row)

        @plsc.parallel_loop(0, CR * VPR, unroll=8)
        def _(t):
            rr = jax.lax.shift_right_logical(t, 5)
            cc = jnp.bitwise_and(t, VPR - 1)
            v = buf[rr, pl.ds(cc * L, L)]
            bits = lax.bitcast_convert_type(v, jnp.int32)
            hi = jax.lax.shift_right_logical(bits, 16)
            lo = jnp.bitwise_and(bits, 0xFFFF)
            plsc.addupdate_scatter(hist, [lo], w, mask=hi == b1)

    _scan_rows(u_hbm, wid * ROWS_PW, buf0, buf1, sem0, sem1, process)
    pltpu.sync_copy(hist, out_hbm.at[wid])


@functools.cache
def _hist1():
    return pl.kernel(
        _hist1_body,
        out_type=jax.ShapeDtypeStruct((NW, NB1), jnp.int32),
        mesh=_sc_mesh(),
        scratch_types=[
            pltpu.VMEM((CR, BLK), jnp.float32),
            pltpu.VMEM((CR, BLK), jnp.float32),
            pltpu.VMEM((NB1,), jnp.int32),
            pltpu.SemaphoreType.DMA,
            pltpu.SemaphoreType.DMA,
        ],
        compiler_params=pltpu.CompilerParams(needs_layout_passes=False),
    )


@functools.cache
def _hist2():
    return pl.kernel(
        _hist2_body,
        out_type=jax.ShapeDtypeStruct((NW, NB2), jnp.int32),
        mesh=_sc_mesh(),
        scratch_types=[
            pltpu.VMEM((CR, BLK), jnp.float32),
            pltpu.VMEM((CR, BLK), jnp.float32),
            pltpu.VMEM((NB2,), jnp.int32),
            pltpu.VMEM((L,), jnp.int32),
            pltpu.SemaphoreType.DMA,
            pltpu.SemaphoreType.DMA,
        ],
        compiler_params=pltpu.CompilerParams(needs_layout_passes=False),
    )


def _prep_body(x_ref, win_ref, xn_ref, h_ref, hb_ref):
    x = x_ref[...]
    nrm = jnp.sqrt(jnp.sum(x * x, axis=1, keepdims=True))
    xn_ref[...] = x / jnp.maximum(nrm, 1e-8)
    h = lax.dot_general(x, win_ref[...], (((1,), (1,)), ((), ())),
                        preferred_element_type=jnp.float32)
    h = jnp.maximum(h, 0.0)
    h_ref[...] = h
    hb_ref[...] = h.astype(jnp.bfloat16)


def _sim_body(xi_ref, xj_ref, sim_ref):
    sim_ref[...] = jnp.abs(
        lax.dot_general(xi_ref[...], xj_ref[...], (((1,), (1,)), ((), ())),
                        preferred_element_type=jnp.float32))


def _sage_body(eps_ref, u_ref, hba_ref, hbb_ref, hfull_ref, wl_ref, bl_ref,
               wr_ref, wo_ref, bo_ref, out_ref, acc_ref, cnt_ref):
    t = pl.program_id(0)
    a = _tri_a(t)
    b = t - a * (2 * NBK + 1 - a) // 2 + a

    @pl.when(t == 0)
    def _():
        acc_ref[...] = jnp.zeros_like(acc_ref)
        cnt_ref[...] = jnp.zeros_like(cnt_ref)

    eps = eps_ref[0, 0]
    mask = u_ref[...] >= eps
    a16 = mask.astype(jnp.bfloat16)

    acc_ref[pl.ds(a * BLK, BLK), :] += lax.dot_general(
        a16, hbb_ref[...], (((1,), (0,)), ((), ())),
        preferred_element_type=jnp.float32)
    csum = jnp.sum(mask.astype(jnp.float32), axis=1, keepdims=True)
    cnt_ref[pl.ds(a * BLK, BLK), :] += jnp.broadcast_to(
        csum, (BLK, cnt_ref.shape[1]))

    @pl.when(b != a)
    def _():
        acc_ref[pl.ds(b * BLK, BLK), :] += lax.dot_general(
            a16, hba_ref[...], (((0,), (0,)), ((), ())),
            preferred_element_type=jnp.float32)
        ones = jnp.ones((BLK, cnt_ref.shape[1]), jnp.bfloat16)
        cnt_ref[pl.ds(b * BLK, BLK), :] += lax.dot_general(
            a16, ones, (((0,), (0,)), ((), ())),
            preferred_element_type=jnp.float32)

    @pl.when(t == pl.num_programs(0) - 1)
    def _():
        acc = acc_ref[...]
        cnt = cnt_ref[...][:, :1]
        mean = acc / jnp.maximum(cnt, 1.0)
        h = hfull_ref[...]
        z = (lax.dot_general(mean, wl_ref[...], (((1,), (1,)), ((), ())),
                             preferred_element_type=jnp.float32)
             + bl_ref[...]
             + lax.dot_general(h, wr_ref[...], (((1,), (1,)), ((), ())),
                               preferred_element_type=jnp.float32))
        z = jnp.maximum(z, 0.0)
        o = lax.dot_general(z, wo_ref[...], (((1,), (1,)), ((), ())),
                            preferred_element_type=jnp.float32) + bo_ref[...]
        m = jnp.max(o, axis=1, keepdims=True)
        e = jnp.exp(o - m)
        out_ref[...] = e / jnp.sum(e, axis=1, keepdims=True)


def _prep_call(x, W_in):
    return pl.pallas_call(
        _prep_body,
        grid=(NBK,),
        in_specs=[
            pl.BlockSpec((BLK, DD), lambda i: (i, 0)),
            pl.BlockSpec((HH, DD), lambda i: (0, 0)),
        ],
        out_specs=[
            pl.BlockSpec((BLK, DD), lambda i: (i, 0)),
            pl.BlockSpec((BLK, HH), lambda i: (i, 0)),
            pl.BlockSpec((BLK, HH), lambda i: (i, 0)),
        ],
        out_shape=[
            jax.ShapeDtypeStruct((NN, DD), jnp.float32),
            jax.ShapeDtypeStruct((NN, HH), jnp.float32),
            jax.ShapeDtypeStruct((NN, HH), jnp.bfloat16),
        ],
    )(x, W_in)


def _sim_call(xn):
    def imap_i(t):
        return (_tri_a(t), 0)

    def imap_j(t):
        a = _tri_a(t)
        return (t - a * (2 * NBK + 1 - a) // 2 + a, 0)

    return pl.pallas_call(
        _sim_body,
        grid=(NT,),
        in_specs=[
            pl.BlockSpec((BLK, DD), imap_i),
            pl.BlockSpec((BLK, DD), imap_j),
        ],
        out_specs=pl.BlockSpec((BLK, BLK), lambda t: (t, 0)),
        out_shape=jax.ShapeDtypeStruct((UROWS, BLK), jnp.float32),
    )(xn, xn)


def _select_eps(h1, h2_fn):
    c1 = jnp.sum(h1, axis=0)
    cum1 = jnp.cumsum(c1)
    b1 = jnp.argmax(cum1 >= (K_RANK + 1)).astype(jnp.int32)
    rank2 = K_RANK - (cum1[b1] - c1[b1])
    h2 = h2_fn(jnp.full((L,), b1, jnp.int32))
    c2 = jnp.sum(h2, axis=0)
    cum2 = jnp.cumsum(c2)
    b2 = jnp.argmax(cum2 >= (rank2 + 1)).astype(jnp.int32)
    eps_bits = jnp.bitwise_or(jnp.left_shift(b1, 16), b2)
    return lax.bitcast_convert_type(eps_bits, jnp.float32)


def _sage_call(eps, u, h, hb, Wl, bl, Wr, Wo, bo):
    def imap_a(t):
        return (_tri_a(t), 0)

    def imap_b(t):
        a = _tri_a(t)
        return (t - a * (2 * NBK + 1 - a) // 2 + a, 0)

    return pl.pallas_call(
        _sage_body,
        grid=(NT,),
        in_specs=[
            pl.BlockSpec(memory_space=pltpu.SMEM),
            pl.BlockSpec((BLK, BLK), lambda t: (t, 0)),
            pl.BlockSpec((BLK, HH), imap_a),
            pl.BlockSpec((BLK, HH), imap_b),
            pl.BlockSpec((NN, HH), lambda t: (0, 0)),
            pl.BlockSpec((HH, HH), lambda t: (0, 0)),
            pl.BlockSpec((1, HH), lambda t: (0, 0)),
            pl.BlockSpec((HH, HH), lambda t: (0, 0)),
            pl.BlockSpec((OUT, HH), lambda t: (0, 0)),
            pl.BlockSpec((1, OUT), lambda t: (0, 0)),
        ],
        out_specs=pl.BlockSpec((NN, OUT), lambda t: (0, 0)),
        out_shape=jax.ShapeDtypeStruct((NN, OUT), jnp.float32),
        scratch_shapes=[
            pltpu.VMEM((NN, HH), jnp.float32),
            pltpu.VMEM((NN, 128), jnp.float32),
        ],
    )(eps, u, hb, hb, h, Wl, bl.reshape(1, HH), Wr, Wo, bo.reshape(1, OUT))


def kernel(x, W_in, Wl, bl, Wr, Wo, bo):
    xn, h, hb = _prep_call(x, W_in)
    u = _sim_call(xn)
    h1 = _hist1()(u)
    eps = _select_eps(h1, lambda b1v: _hist2()(u, b1v))
    return _sage_call(eps.reshape(1, 1), u, h, hb, Wl, bl, Wr, Wo, bo)


# speedup vs baseline: 1.1466x; 1.0018x over previous
import functools

import jax
import jax.numpy as jnp
from jax import lax
from jax.experimental import pallas as pl
from jax.experimental.pallas import tpu as pltpu
from jax.experimental.pallas import tpu_sc as plsc

NN = 4096
DD = 256
HH = 256
OUT = 64
BLK = 512
NBK = NN // BLK
NT = NBK * (NBK + 1) // 2
UROWS = NT * BLK
TOT = NN * NN
K_RANK = int(round(0.99 * (TOT - 1)))


def _tri_base(a):
    return a * (2 * NBK + 1 - a) // 2


def _tri_a(t):
    a = jnp.int32(0)
    for k in range(1, NBK):
        a = a + (t >= _tri_base(k)).astype(jnp.int32)
    return a


NC, NS, L = 2, 16, 16
NW = NC * NS
ROWS_PW = UROWS // NW
CR = 32
VPR = BLK // L
NB1 = 1 << 15
NB2 = 1 << 16


@functools.cache
def _sc_mesh():
    return plsc.VectorSubcoreMesh(
        core_axis_name="c", subcore_axis_name="s",
        num_cores=NC, num_subcores=NS)


def _zero_hist(hist, nb):
    zeros = jnp.zeros((L,), jnp.int32)

    def body(t, carry):
        hist[pl.ds(t * L, L)] = zeros
        return carry

    lax.fori_loop(0, nb // L, body, 0)


_DIAG_TS = tuple(_tri_base(a) for a in range(NBK))


def _chunk_weight(row):
    tt = jax.lax.shift_right_logical(row, 9)
    isd = jnp.int32(0)
    for t in _DIAG_TS:
        isd = isd | (tt == t).astype(jnp.int32)
    return jnp.broadcast_to(2 - isd, (L,))


def _scan_rows(u_hbm, row0, buf0, buf1, sem0, sem1, process):
    nch = ROWS_PW // CR

    def start(c, buf, sem):
        pltpu.make_async_copy(
            u_hbm.at[pl.ds(row0 + c * CR, CR)], buf, sem).start()

    def wait(buf, sem):
        pltpu.make_async_copy(u_hbm.at[pl.ds(row0, CR)], buf, sem).wait()

    start(0, buf0, sem0)

    def outer(g, carry):
        c0 = 2 * g
        start(c0 + 1, buf1, sem1)
        wait(buf0, sem0)
        process(buf0, row0 + c0 * CR)

        @pl.when(c0 + 2 < nch)
        def _():
            start(c0 + 2, buf0, sem0)

        wait(buf1, sem1)
        process(buf1, row0 + (c0 + 1) * CR)
        return carry

    lax.fori_loop(0, nch // 2, outer, 0)


def _hist1_body(u_hbm, out_hbm, buf0, buf1, hist, sem0, sem1):
    wid = lax.axis_index("s") * NC + lax.axis_index("c")
    _zero_hist(hist, NB1)

    def process(buf, row):
        w = _chunk_weight(row)

        @plsc.parallel_loop(0, CR * VPR, unroll=16)
        def _(t):
            rr = jax.lax.shift_right_logical(t, 5)
            cc = jnp.bitwise_and(t, VPR - 1)
            v = buf[rr, pl.ds(cc * L, L)]
            bits = lax.bitcast_convert_type(v, jnp.int32)
            idx = jax.lax.shift_right_logical(bits, 16)
            plsc.addupdate_scatter(hist, [idx], w)

    _scan_rows(u_hbm, wid * ROWS_PW, buf0, buf1, sem0, sem1, process)
    pltpu.sync_copy(hist, out_hbm.at[wid])


def _hist2_body(u_hbm, b1_hbm, out_hbm, buf0, buf1, hist, b1v, sem0, sem1):
    wid = lax.axis_index("s") * NC + lax.axis_index("c")
    _zero_hist(hist, NB2)
    pltpu.sync_copy(b1_hbm, b1v)
    b1 = b1v[...]

    def process(buf, row):
        w = _chunk_weight(row)

        @plsc.parallel_loop(0, CR * VPR, unroll=16)
        def _(t):
            rr = jax.lax.shift_right_logical(t, 5)
            cc = jnp.bitwise_and(t, VPR - 1)
            v = buf[rr, pl.ds(cc * L, L)]
            bits = lax.bitcast_convert_type(v, jnp.int32)
            hi = jax.lax.shift_right_logical(bits, 16)
            lo = jnp.bitwise_and(bits, 0xFFFF)
            plsc.addupdate_scatter(hist, [lo], w, mask=hi == b1)

    _scan_rows(u_hbm, wid * ROWS_PW, buf0, buf1, sem0, sem1, process)
    pltpu.sync_copy(hist, out_hbm.at[wid])


@functools.cache
def _hist1():
    return pl.kernel(
        _hist1_body,
        out_type=jax.ShapeDtypeStruct((NW, NB1), jnp.int32),
        mesh=_sc_mesh(),
        scratch_types=[
            pltpu.VMEM((CR, BLK), jnp.float32),
            pltpu.VMEM((CR, BLK), jnp.float32),
            pltpu.VMEM((NB1,), jnp.int32),
            pltpu.SemaphoreType.DMA,
            pltpu.SemaphoreType.DMA,
        ],
        compiler_params=pltpu.CompilerParams(needs_layout_passes=False),
    )


@functools.cache
def _hist2():
    return pl.kernel(
        _hist2_body,
        out_type=jax.ShapeDtypeStruct((NW, NB2), jnp.int32),
        mesh=_sc_mesh(),
        scratch_types=[
            pltpu.VMEM((CR, BLK), jnp.float32),
            pltpu.VMEM((CR, BLK), jnp.float32),
            pltpu.VMEM((NB2,), jnp.int32),
            pltpu.VMEM((L,), jnp.int32),
            pltpu.SemaphoreType.DMA,
            pltpu.SemaphoreType.DMA,
        ],
        compiler_params=pltpu.CompilerParams(needs_layout_passes=False),
    )


def _prep_body(x_ref, win_ref, xn_ref, h_ref, hb_ref):
    x = x_ref[...]
    nrm = jnp.sqrt(jnp.sum(x * x, axis=1, keepdims=True))
    xn_ref[...] = x / jnp.maximum(nrm, 1e-8)
    h = lax.dot_general(x, win_ref[...], (((1,), (1,)), ((), ())),
                        preferred_element_type=jnp.float32)
    h = jnp.maximum(h, 0.0)
    h_ref[...] = h
    hb_ref[...] = h.astype(jnp.bfloat16)


def _sim_body(xi_ref, xj_ref, sim_ref):
    sim_ref[...] = jnp.abs(
        lax.dot_general(xi_ref[...], xj_ref[...], (((1,), (1,)), ((), ())),
                        preferred_element_type=jnp.float32))


def _sage_body(eps_ref, u_ref, hba_ref, hbb_ref, hfull_ref, wl_ref, bl_ref,
               wr_ref, wo_ref, bo_ref, out_ref, acc_ref, cnt_ref):
    t = pl.program_id(0)
    a = _tri_a(t)
    b = t - a * (2 * NBK + 1 - a) // 2 + a

    @pl.when(t == 0)
    def _():
        acc_ref[...] = jnp.zeros_like(acc_ref)
        cnt_ref[...] = jnp.zeros_like(cnt_ref)

    eps = eps_ref[0, 0]
    mask = u_ref[...] >= eps
    a16 = mask.astype(jnp.bfloat16)

    acc_ref[pl.ds(a * BLK, BLK), :] += lax.dot_general(
        a16, hbb_ref[...], (((1,), (0,)), ((), ())),
        preferred_element_type=jnp.float32)
    csum = jnp.sum(mask.astype(jnp.float32), axis=1, keepdims=True)
    cnt_ref[pl.ds(a * BLK, BLK), :] += jnp.broadcast_to(
        csum, (BLK, cnt_ref.shape[1]))

    @pl.when(b != a)
    def _():
        acc_ref[pl.ds(b * BLK, BLK), :] += lax.dot_general(
            a16, hba_ref[...], (((0,), (0,)), ((), ())),
            preferred_element_type=jnp.float32)
        ones = jnp.ones((BLK, cnt_ref.shape[1]), jnp.bfloat16)
        cnt_ref[pl.ds(b * BLK, BLK), :] += lax.dot_general(
            a16, ones, (((0,), (0,)), ((), ())),
            preferred_element_type=jnp.float32)

    @pl.when(t == pl.num_programs(0) - 1)
    def _():
        acc = acc_ref[...]
        cnt = cnt_ref[...][:, :1]
        mean = acc / jnp.maximum(cnt, 1.0)
        h = hfull_ref[...]
        z = (lax.dot_general(mean, wl_ref[...], (((1,), (1,)), ((), ())),
                             preferred_element_type=jnp.float32)
             + bl_ref[...]
             + lax.dot_general(h, wr_ref[...], (((1,), (1,)), ((), ())),
                               preferred_element_type=jnp.float32))
        z = jnp.maximum(z, 0.0)
        o = lax.dot_general(z, wo_ref[...], (((1,), (1,)), ((), ())),
                            preferred_element_type=jnp.float32) + bo_ref[...]
        m = jnp.max(o, axis=1, keepdims=True)
        e = jnp.exp(o - m)
        out_ref[...] = e / jnp.sum(e, axis=1, keepdims=True)


def _prep_call(x, W_in):
    return pl.pallas_call(
        _prep_body,
        grid=(NBK,),
        in_specs=[
            pl.BlockSpec((BLK, DD), lambda i: (i, 0)),
            pl.BlockSpec((HH, DD), lambda i: (0, 0)),
        ],
        out_specs=[
            pl.BlockSpec((BLK, DD), lambda i: (i, 0)),
            pl.BlockSpec((BLK, HH), lambda i: (i, 0)),
            pl.BlockSpec((BLK, HH), lambda i: (i, 0)),
        ],
        out_shape=[
            jax.ShapeDtypeStruct((NN, DD), jnp.float32),
            jax.ShapeDtypeStruct((NN, HH), jnp.float32),
            jax.ShapeDtypeStruct((NN, HH), jnp.bfloat16),
        ],
    )(x, W_in)


def _sim_call(xn):
    def imap_i(t):
        return (_tri_a(t), 0)

    def imap_j(t):
        a = _tri_a(t)
        return (t - a * (2 * NBK + 1 - a) // 2 + a, 0)

    return pl.pallas_call(
        _sim_body,
        grid=(NT,),
        in_specs=[
            pl.BlockSpec((BLK, DD), imap_i),
            pl.BlockSpec((BLK, DD), imap_j),
        ],
        out_specs=pl.BlockSpec((BLK, BLK), lambda t: (t, 0)),
        out_shape=jax.ShapeDtypeStruct((UROWS, BLK), jnp.float32),
    )(xn, xn)


def _select_eps(h1, h2_fn):
    c1 = jnp.sum(h1, axis=0)
    cum1 = jnp.cumsum(c1)
    b1 = jnp.argmax(cum1 >= (K_RANK + 1)).astype(jnp.int32)
    rank2 = K_RANK - (cum1[b1] - c1[b1])
    h2 = h2_fn(jnp.full((L,), b1, jnp.int32))
    c2 = jnp.sum(h2, axis=0)
    cum2 = jnp.cumsum(c2)
    b2 = jnp.argmax(cum2 >= (rank2 + 1)).astype(jnp.int32)
    eps_bits = jnp.bitwise_or(jnp.left_shift(b1, 16), b2)
    return lax.bitcast_convert_type(eps_bits, jnp.float32)


def _sage_call(eps, u, h, hb, Wl, bl, Wr, Wo, bo):
    def imap_a(t):
        return (_tri_a(t), 0)

    def imap_b(t):
        a = _tri_a(t)
        return (t - a * (2 * NBK + 1 - a) // 2 + a, 0)

    return pl.pallas_call(
        _sage_body,
        grid=(NT,),
        in_specs=[
            pl.BlockSpec(memory_space=pltpu.SMEM),
            pl.BlockSpec((BLK, BLK), lambda t: (t, 0)),
            pl.BlockSpec((BLK, HH), imap_a),
            pl.BlockSpec((BLK, HH), imap_b),
            pl.BlockSpec((NN, HH), lambda t: (0, 0)),
            pl.BlockSpec((HH, HH), lambda t: (0, 0)),
            pl.BlockSpec((1, HH), lambda t: (0, 0)),
            pl.BlockSpec((HH, HH), lambda t: (0, 0)),
            pl.BlockSpec((OUT, HH), lambda t: (0, 0)),
            pl.BlockSpec((1, OUT), lambda t: (0, 0)),
        ],
        out_specs=pl.BlockSpec((NN, OUT), lambda t: (0, 0)),
        out_shape=jax.ShapeDtypeStruct((NN, OUT), jnp.float32),
        scratch_shapes=[
            pltpu.VMEM((NN, HH), jnp.float32),
            pltpu.VMEM((NN, 128), jnp.float32),
        ],
    )(eps, u, hb, hb, h, Wl, bl.reshape(1, HH), Wr, Wo, bo.reshape(1, OUT))


def kernel(x, W_in, Wl, bl, Wr, Wo, bo):
    xn, h, hb = _prep_call(x, W_in)
    u = _sim_call(xn)
    h1 = _hist1()(u)
    eps = _select_eps(h1, lambda b1v: _hist2()(u, b1v))
    return _sage_call(eps.reshape(1, 1), u, h, hb, Wl, bl, Wr, Wo, bo)
